# baseline (speedup 1.0000x reference)
import functools

import numpy as np
import jax
import jax.numpy as jnp
from jax import lax
from jax.experimental import pallas as pl
from jax.experimental.pallas import tpu as pltpu
from jax.experimental.pallas import tpu_sc as plsc

_EPS = 1e-5


def _mm_body(x_ref, w_ref, o_ref):
    r = jnp.dot(x_ref[...], w_ref[...], preferred_element_type=jnp.float32, precision=lax.Precision.HIGHEST)
    fh = o_ref.shape[2]
    o_ref[0] = r[:, :fh]
    o_ref[1] = r[:, fh:]


def _matmul_split(x, w):
    return pl.pallas_call(
        _mm_body,
        out_shape=jax.ShapeDtypeStruct((2, x.shape[0], w.shape[1] // 2),
                                       jnp.float32),
    )(x, w)


def _mid_body(ys_ref, agg_ref, b1_ref, g_ref, beta_ref, w2_ref, b2_ref,
              w1n_ref, o_ref):
    n = ys_ref.shape[1]
    y = jnp.concatenate([ys_ref[0], ys_ref[1]], axis=1)
    agg = jnp.concatenate([agg_ref[:n], agg_ref[n:]], axis=1)
    z = y + agg + b1_ref[...]
    mu = jnp.mean(z, axis=0, keepdims=True)
    d = z - mu
    var = jnp.mean(d * d, axis=0, keepdims=True)
    zn = g_ref[...] * d * lax.rsqrt(var + _EPS) + beta_ref[...]
    a = jnp.maximum(zn, 0.0)
    h = jnp.maximum(
        jnp.dot(a, w2_ref[...], preferred_element_type=jnp.float32, precision=lax.Precision.HIGHEST)
        + b2_ref[...], 0.0)
    o_ref[...] = jnp.dot(h, w1n_ref[...], preferred_element_type=jnp.float32, precision=lax.Precision.HIGHEST)


def _mid(ys, agg, b1, g, beta, w2, b2, w1n):
    return pl.pallas_call(
        _mid_body,
        out_shape=jax.ShapeDtypeStruct((ys.shape[1], w1n.shape[1]),
                                       jnp.float32),
    )(ys, agg, b1, g, beta, w2, b2, w1n)


def _final_body(y_ref, agg_ref, b1_ref, g_ref, beta_ref, w2_ref, b2_ref,
                gids_ref, sf_ref, r_ref, s_ref,
                fc1w_ref, fc1b_ref, bn1g_ref, bn1b_ref,
                fc2w_ref, fc2b_ref, bn2g_ref, bn2b_ref,
                fc3w_ref, fc3b_ref, o_ref):
    n = y_ref.shape[0]
    z = y_ref[...] + b1_ref[...]
    for p in range(agg_ref.shape[0] // n):
        z = z + agg_ref[p * n:(p + 1) * n, :]
    mu = jnp.mean(z, axis=0, keepdims=True)
    d = z - mu
    var = jnp.mean(d * d, axis=0, keepdims=True)
    zn = g_ref[...] * d * lax.rsqrt(var + _EPS) + beta_ref[...]
    a = jnp.maximum(zn, 0.0)
    h = jnp.maximum(
        jnp.dot(a, w2_ref[...], preferred_element_type=jnp.float32, precision=lax.Precision.HIGHEST)
        + b2_ref[...], 0.0)

    b_graphs = o_ref.shape[0]
    rows = lax.broadcasted_iota(jnp.int32, (b_graphs, n), 0)
    oh = (rows == gids_ref[...]).astype(jnp.float32)
    sums = jnp.dot(oh, h, preferred_element_type=jnp.float32, precision=lax.Precision.HIGHEST)
    counts = jnp.sum(oh, axis=1, keepdims=True)
    hg = sums / jnp.maximum(counts, 1.0)

    gmat = (jnp.dot(hg, r_ref[...], preferred_element_type=jnp.float32, precision=lax.Precision.HIGHEST)
            * jnp.dot(sf_ref[...], s_ref[...],
                      preferred_element_type=jnp.float32, precision=lax.Precision.HIGHEST))

    def bn(v, gg, bb):
        m = jnp.mean(v, axis=0, keepdims=True)
        dd = v - m
        vv = jnp.mean(dd * dd, axis=0, keepdims=True)
        return gg * dd * lax.rsqrt(vv + _EPS) + bb

    o1 = jnp.maximum(
        bn(jnp.dot(gmat, fc1w_ref[...], preferred_element_type=jnp.float32, precision=lax.Precision.HIGHEST)
           + fc1b_ref[...], bn1g_ref[...], bn1b_ref[...]), 0.0)
    o2 = jnp.maximum(
        bn(jnp.dot(o1, fc2w_ref[...], preferred_element_type=jnp.float32, precision=lax.Precision.HIGHEST)
           + fc2b_ref[...], bn2g_ref[...], bn2b_ref[...]), 0.0)
    o_ref[...] = (jnp.dot(o2, fc3w_ref[...],
                          preferred_element_type=jnp.float32, precision=lax.Precision.HIGHEST)
                  + fc3b_ref[...])


def _final(y, agg, b1, g, beta, w2, b2, gids, sf, r, s,
           fc1w, fc1b, bn1g, bn1b, fc2w, fc2b, bn2g, bn2b, fc3w, fc3b):
    b_graphs = sf.shape[0]
    return pl.pallas_call(
        _final_body,
        out_shape=jax.ShapeDtypeStruct((b_graphs, fc3w.shape[1]),
                                       jnp.float32),
    )(y, agg, b1, g, beta, w2, b2, gids, sf, r, s,
      fc1w, fc1b, bn1g, bn1b, fc2w, fc2b, bn2g, bn2b, fc3w, fc3b)


_NS = 16
_WCH = 80


@functools.lru_cache(maxsize=None)
def _make_segsum(n_nodes, feat, n_edges, mode):
    batch, nb = 100, 10
    if mode == "col":
        bpt = n_edges // _NS // batch
    else:
        bpt = n_edges // (2 * _NS) // batch
    nref = bpt // nb
    assert nref * nb == bpt and nref % 2 == 0
    nwchunk = n_nodes // _WCH
    assert nwchunk * _WCH == n_nodes
    wper = -(-nwchunk // _NS)
    mesh = plsc.VectorSubcoreMesh(core_axis_name="c", subcore_axis_name="s",
                                  num_cores=2)
    scratch = [
        pltpu.VMEM((nb, batch), jnp.int32),
        pltpu.VMEM((nb, batch), jnp.int32),
        pltpu.VMEM((nb, batch), jnp.int32),
        pltpu.VMEM((nb, batch), jnp.int32),
        pltpu.VMEM((batch, feat), jnp.float32),
        pltpu.VMEM((batch, feat), jnp.float32),
        pltpu.VMEM((batch, feat), jnp.float32),
        pltpu.VMEM((batch, feat), jnp.float32),
        pltpu.VMEM_SHARED((n_nodes, feat), jnp.float32),
        pltpu.SemaphoreType.DMA,
        pltpu.SemaphoreType.DMA,
        pltpu.SemaphoreType.DMA,
    ]
    @functools.partial(
        pl.kernel, mesh=mesh,
        compiler_params=pltpu.CompilerParams(use_tc_tiling_on_sc=False),
        out_type=jax.ShapeDtypeStruct((2 * n_nodes, feat), jnp.float32),
        scratch_types=scratch,
    )
    def segsum(y_hbm, src_hbm, dst_hbm, zeros_hbm, out_hbm,
               srcA, dstA, srcB, dstB, r0, r1, r2, r3, acc,
               semg, sems, semi):
        c = lax.axis_index("c")
        s = lax.axis_index("s")
        bufs = (r0, r1, r2, r3)
        if mode == "col":
            ysel = y_hbm.at[c]
            base = s * bpt
        else:
            ysel = y_hbm
            base = (c * _NS + s) * bpt

        pltpu.sync_copy(zeros_hbm, r0.at[pl.ds(0, _WCH)])
        for r in range(wper):
            k = s + _NS * r

            @pl.when(k < nwchunk)
            def _():
                pltpu.sync_copy(r0.at[pl.ds(0, _WCH)],
                                acc.at[pl.ds(k * _WCH, _WCH)])
        plsc.subcore_barrier()

        def idx_start(g, sv, dv):
            blk0 = base + g * nb
            pltpu.async_copy(src_hbm.at[pl.ds(blk0, nb)], sv, semi)
            pltpu.async_copy(dst_hbm.at[pl.ds(blk0, nb)], dv, semi)

        def idx_wait(g, sv, dv):
            blk0 = base + g * nb
            pltpu.make_async_copy(src_hbm.at[pl.ds(blk0, nb)], sv,
                                  semi).wait()
            pltpu.make_async_copy(dst_hbm.at[pl.ds(blk0, nb)], dv,
                                  semi).wait()

        def g_start(sv, j):
            pltpu.async_copy(ysel.at[sv.at[j]], bufs[j % 4], semg)

        def g_wait(sv, j):
            pltpu.make_async_copy(ysel.at[sv.at[j]], bufs[j % 4],
                                  semg).wait()

        def s_start(dv, j):
            pltpu.async_copy(bufs[j % 4], acc.at[dv.at[j]], sems, add=True)

        def s_wait(dv, j):
            pltpu.make_async_copy(bufs[j % 4], acc.at[dv.at[j]],
                                  sems).wait()

        def run_refill(sv, dv):
            for j in range(min(3, nb)):
                g_start(sv, j)
            for j in range(nb):
                g_wait(sv, j)
                s_start(dv, j)
                if j >= 1:
                    s_wait(dv, j - 1)
                if j + 3 < nb:
                    g_start(sv, j + 3)
            s_wait(dv, nb - 1)

        idx_start(0, srcA, dstA)

        def body(i, carry):
            g0 = 2 * i
            idx_wait(g0, srcA, dstA)
            idx_start(g0 + 1, srcB, dstB)
            run_refill(srcA, dstA)
            idx_wait(g0 + 1, srcB, dstB)

            @pl.when(g0 + 2 < nref)
            def _():
                idx_start(g0 + 2, srcA, dstA)

            run_refill(srcB, dstB)
            return carry

        lax.fori_loop(0, nref // 2, body, 0)
        plsc.subcore_barrier()

        for r in range(wper):
            k = s + _NS * r

            @pl.when(k < nwchunk)
            def _():
                pltpu.sync_copy(acc.at[pl.ds(k * _WCH, _WCH)],
                                r0.at[pl.ds(0, _WCH)])
                pltpu.sync_copy(r0.at[pl.ds(0, _WCH)],
                                out_hbm.at[pl.ds(c * n_nodes + k * _WCH,
                                                 _WCH)])

    return segsum


def _segsum(y, src2d, dst2d, zeros, mode):
    if mode == "col":
        _, n, f = y.shape
    else:
        n, f = y.shape
    e = src2d.shape[0] * src2d.shape[1]
    return _make_segsum(n, f, e, mode)(y, src2d, dst2d, zeros)


def _pad_cols(a, cols):
    return jnp.pad(a, ((0, 0), (0, cols - a.shape[1])))


def kernel(x, edge_index, graph_ids, self_feat,
           gin1_W1, gin1_b1, gin1_g, gin1_beta, gin1_W2, gin1_b2,
           gin2_W1, gin2_b1, gin2_g, gin2_beta, gin2_W2, gin2_b2,
           fc1_W, fc1_b, bn1_g, bn1_b, fc2_W, fc2_b, bn2_g, bn2_b,
           fc3_W, fc3_b):
    n = x.shape[0]
    f1 = 112
    f2 = 32
    d_self = self_feat.shape[1]
    d_mid = gin2_W2.shape[1]
    d_hid = gin1_W2.shape[1]

    def padv(v, cols, fill=0.0):
        return jnp.pad(v, (0, cols - v.shape[0]),
                       constant_values=fill).reshape(1, cols)

    w1p = _pad_cols(gin1_W1, f1)
    b1p = padv(gin1_b1, f1)
    g1p = padv(gin1_g, f1, 1.0)
    beta1p = padv(gin1_beta, f1)
    w2p = jnp.pad(gin1_W2, ((0, f1 - d_hid), (0, f1 - d_hid)))
    b2p = padv(gin1_b2, f1)
    w12p = jnp.pad(gin2_W1, ((0, f1 - d_hid), (0, f2 - d_mid)))

    b1p2 = padv(gin2_b1, f2)
    g2p = padv(gin2_g, f2, 1.0)
    beta2p = padv(gin2_beta, f2)
    w22p = jnp.pad(gin2_W2, ((0, f2 - d_mid), (0, f2 - d_mid)))
    b2p2 = padv(gin2_b2, f2)

    rmat = np.zeros((f2, d_mid * d_self), np.float32)
    smat = np.zeros((d_self, d_mid * d_self), np.float32)
    for i in range(d_mid):
        for j in range(d_self):
            rmat[i, i * d_self + j] = 1.0
            smat[j, i * d_self + j] = 1.0
    rmat = jnp.asarray(rmat)
    smat = jnp.asarray(smat)

    src2d = edge_index[0].reshape(-1, 100)
    dst2d = edge_index[1].reshape(-1, 100)
    zeros1 = jnp.zeros((_WCH, f1 // 2), jnp.float32)
    zeros2 = jnp.zeros((_WCH, f2), jnp.float32)
    gids = graph_ids.reshape(1, n)

    ystack = _matmul_split(x, w1p)
    acols = _segsum(ystack, src2d, dst2d, zeros1, "col")
    y2 = _mid(ystack, acols, b1p, g1p, beta1p, w2p, b2p, w12p)
    agg2 = _segsum(y2, src2d, dst2d, zeros2, "edge")
    out = _final(y2, agg2, b1p2, g2p, beta2p, w22p, b2p2,
                 gids, self_feat, rmat, smat,
                 fc1_W, fc1_b.reshape(1, -1),
                 bn1_g.reshape(1, -1), bn1_b.reshape(1, -1),
                 fc2_W, fc2_b.reshape(1, -1),
                 bn2_g.reshape(1, -1), bn2_b.reshape(1, -1),
                 fc3_W, fc3_b.reshape(1, -1))
    return out


# speedup vs baseline: 1.1470x; 1.1470x over previous
import functools

import numpy as np
import jax
import jax.numpy as jnp
from jax import lax
from jax.experimental import pallas as pl
from jax.experimental.pallas import tpu as pltpu
from jax.experimental.pallas import tpu_sc as plsc

_EPS = 1e-5


def _mm_body(x_ref, w_ref, o_ref):
    r = jnp.dot(x_ref[...], w_ref[...], preferred_element_type=jnp.float32)
    fh = o_ref.shape[2]
    o_ref[0] = r[:, :fh]
    o_ref[1] = r[:, fh:]


def _matmul_split(x, w):
    return pl.pallas_call(
        _mm_body,
        out_shape=jax.ShapeDtypeStruct((2, x.shape[0], w.shape[1] // 2),
                                       jnp.float32),
    )(x, w)


def _mid_body(ys_ref, agg_ref, b1_ref, g_ref, beta_ref, w2_ref, b2_ref,
              w1n_ref, o_ref):
    n = ys_ref.shape[1]
    y = jnp.concatenate([ys_ref[0], ys_ref[1]], axis=1)
    agg = jnp.concatenate([agg_ref[:n], agg_ref[n:]], axis=1)
    z = y + agg + b1_ref[...]
    mu = jnp.mean(z, axis=0, keepdims=True)
    d = z - mu
    var = jnp.mean(d * d, axis=0, keepdims=True)
    zn = g_ref[...] * d * lax.rsqrt(var + _EPS) + beta_ref[...]
    a = jnp.maximum(zn, 0.0)
    h = jnp.maximum(
        jnp.dot(a, w2_ref[...], preferred_element_type=jnp.float32)
        + b2_ref[...], 0.0)
    o_ref[...] = jnp.dot(h, w1n_ref[...], preferred_element_type=jnp.float32)


def _mid(ys, agg, b1, g, beta, w2, b2, w1n):
    return pl.pallas_call(
        _mid_body,
        out_shape=jax.ShapeDtypeStruct((ys.shape[1], w1n.shape[1]),
                                       jnp.float32),
    )(ys, agg, b1, g, beta, w2, b2, w1n)


def _final_body(y_ref, agg_ref, b1_ref, g_ref, beta_ref, w2_ref, b2_ref,
                gids_ref, sf_ref, r_ref, s_ref,
                fc1w_ref, fc1b_ref, bn1g_ref, bn1b_ref,
                fc2w_ref, fc2b_ref, bn2g_ref, bn2b_ref,
                fc3w_ref, fc3b_ref, o_ref):
    n = y_ref.shape[0]
    z = y_ref[...] + b1_ref[...]
    for p in range(agg_ref.shape[0] // n):
        z = z + agg_ref[p * n:(p + 1) * n, :]
    mu = jnp.mean(z, axis=0, keepdims=True)
    d = z - mu
    var = jnp.mean(d * d, axis=0, keepdims=True)
    zn = g_ref[...] * d * lax.rsqrt(var + _EPS) + beta_ref[...]
    a = jnp.maximum(zn, 0.0)
    h = jnp.maximum(
        jnp.dot(a, w2_ref[...], preferred_element_type=jnp.float32)
        + b2_ref[...], 0.0)

    b_graphs = o_ref.shape[0]
    rows = lax.broadcasted_iota(jnp.int32, (b_graphs, n), 0)
    oh = (rows == gids_ref[...]).astype(jnp.float32)
    sums = jnp.dot(oh, h, preferred_element_type=jnp.float32)
    counts = jnp.sum(oh, axis=1, keepdims=True)
    hg = sums / jnp.maximum(counts, 1.0)

    gmat = (jnp.dot(hg, r_ref[...], preferred_element_type=jnp.float32)
            * jnp.dot(sf_ref[...], s_ref[...],
                      preferred_element_type=jnp.float32))

    def bn(v, gg, bb):
        m = jnp.mean(v, axis=0, keepdims=True)
        dd = v - m
        vv = jnp.mean(dd * dd, axis=0, keepdims=True)
        return gg * dd * lax.rsqrt(vv + _EPS) + bb

    o1 = jnp.maximum(
        bn(jnp.dot(gmat, fc1w_ref[...], preferred_element_type=jnp.float32)
           + fc1b_ref[...], bn1g_ref[...], bn1b_ref[...]), 0.0)
    o2 = jnp.maximum(
        bn(jnp.dot(o1, fc2w_ref[...], preferred_element_type=jnp.float32)
           + fc2b_ref[...], bn2g_ref[...], bn2b_ref[...]), 0.0)
    o_ref[...] = (jnp.dot(o2, fc3w_ref[...],
                          preferred_element_type=jnp.float32)
                  + fc3b_ref[...])


def _final(y, agg, b1, g, beta, w2, b2, gids, sf, r, s,
           fc1w, fc1b, bn1g, bn1b, fc2w, fc2b, bn2g, bn2b, fc3w, fc3b):
    b_graphs = sf.shape[0]
    return pl.pallas_call(
        _final_body,
        out_shape=jax.ShapeDtypeStruct((b_graphs, fc3w.shape[1]),
                                       jnp.float32),
    )(y, agg, b1, g, beta, w2, b2, gids, sf, r, s,
      fc1w, fc1b, bn1g, bn1b, fc2w, fc2b, bn2g, bn2b, fc3w, fc3b)


_NS = 16
_WCH = 80


@functools.lru_cache(maxsize=None)
def _make_segsum(n_nodes, feat, n_edges, mode):
    batch, nb = 100, 10
    if mode == "col":
        bpt = n_edges // _NS // batch
    else:
        bpt = n_edges // (2 * _NS) // batch
    nref = bpt // nb
    assert nref * nb == bpt and nref % 2 == 0
    nwchunk = n_nodes // _WCH
    assert nwchunk * _WCH == n_nodes
    wper = -(-nwchunk // _NS)
    mesh = plsc.VectorSubcoreMesh(core_axis_name="c", subcore_axis_name="s",
                                  num_cores=2)
    scratch = [
        pltpu.VMEM((nb, batch), jnp.int32),
        pltpu.VMEM((nb, batch), jnp.int32),
        pltpu.VMEM((nb, batch), jnp.int32),
        pltpu.VMEM((nb, batch), jnp.int32),
        pltpu.VMEM((batch, feat), jnp.float32),
        pltpu.VMEM((batch, feat), jnp.float32),
        pltpu.VMEM((batch, feat), jnp.float32),
        pltpu.VMEM((batch, feat), jnp.float32),
        pltpu.VMEM_SHARED((n_nodes, feat), jnp.float32),
        pltpu.SemaphoreType.DMA,
        pltpu.SemaphoreType.DMA,
        pltpu.SemaphoreType.DMA,
    ]
    @functools.partial(
        pl.kernel, mesh=mesh,
        compiler_params=pltpu.CompilerParams(use_tc_tiling_on_sc=False,
                                             skip_device_barrier=True),
        out_type=jax.ShapeDtypeStruct((2 * n_nodes, feat), jnp.float32),
        scratch_types=scratch,
    )
    def segsum(y_hbm, src_hbm, dst_hbm, zeros_hbm, out_hbm,
               srcA, dstA, srcB, dstB, r0, r1, r2, r3, acc,
               semg, sems, semi):
        c = lax.axis_index("c")
        s = lax.axis_index("s")
        bufs = (r0, r1, r2, r3)
        if mode == "col":
            ysel = y_hbm.at[c]
            base = s * bpt
        else:
            ysel = y_hbm
            base = (c * _NS + s) * bpt

        pltpu.sync_copy(zeros_hbm, r0.at[pl.ds(0, _WCH)])
        for r in range(wper):
            k = s + _NS * r

            @pl.when(k < nwchunk)
            def _():
                pltpu.sync_copy(r0.at[pl.ds(0, _WCH)],
                                acc.at[pl.ds(k * _WCH, _WCH)])
        plsc.subcore_barrier()

        def idx_start(g, sv, dv):
            blk0 = base + g * nb
            pltpu.async_copy(src_hbm.at[pl.ds(blk0, nb)], sv, semi)
            pltpu.async_copy(dst_hbm.at[pl.ds(blk0, nb)], dv, semi)

        def idx_wait(g, sv, dv):
            blk0 = base + g * nb
            pltpu.make_async_copy(src_hbm.at[pl.ds(blk0, nb)], sv,
                                  semi).wait()
            pltpu.make_async_copy(dst_hbm.at[pl.ds(blk0, nb)], dv,
                                  semi).wait()

        def g_start(sv, j):
            pltpu.async_copy(ysel.at[sv.at[j]], bufs[j % 4], semg)

        def g_wait(sv, j):
            pltpu.make_async_copy(ysel.at[sv.at[j]], bufs[j % 4],
                                  semg).wait()

        def s_start(dv, j):
            pltpu.async_copy(bufs[j % 4], acc.at[dv.at[j]], sems, add=True)

        def s_wait(dv, j):
            pltpu.make_async_copy(bufs[j % 4], acc.at[dv.at[j]],
                                  sems).wait()

        def run_refill(sv, dv):
            for j in range(min(3, nb)):
                g_start(sv, j)
            for j in range(nb):
                g_wait(sv, j)
                s_start(dv, j)
                if j >= 1:
                    s_wait(dv, j - 1)
                if j + 3 < nb:
                    g_start(sv, j + 3)
            s_wait(dv, nb - 1)

        idx_start(0, srcA, dstA)

        def body(i, carry):
            g0 = 2 * i
            idx_wait(g0, srcA, dstA)
            idx_start(g0 + 1, srcB, dstB)
            run_refill(srcA, dstA)
            idx_wait(g0 + 1, srcB, dstB)

            @pl.when(g0 + 2 < nref)
            def _():
                idx_start(g0 + 2, srcA, dstA)

            run_refill(srcB, dstB)
            return carry

        lax.fori_loop(0, nref // 2, body, 0)
        plsc.subcore_barrier()

        for r in range(wper):
            k = s + _NS * r

            @pl.when(k < nwchunk)
            def _():
                pltpu.sync_copy(acc.at[pl.ds(k * _WCH, _WCH)],
                                r0.at[pl.ds(0, _WCH)])
                pltpu.sync_copy(r0.at[pl.ds(0, _WCH)],
                                out_hbm.at[pl.ds(c * n_nodes + k * _WCH,
                                                 _WCH)])

    return segsum


def _segsum(y, src2d, dst2d, zeros, mode):
    if mode == "col":
        _, n, f = y.shape
    else:
        n, f = y.shape
    e = src2d.shape[0] * src2d.shape[1]
    return _make_segsum(n, f, e, mode)(y, src2d, dst2d, zeros)


def _pad_cols(a, cols):
    return jnp.pad(a, ((0, 0), (0, cols - a.shape[1])))


def kernel(x, edge_index, graph_ids, self_feat,
           gin1_W1, gin1_b1, gin1_g, gin1_beta, gin1_W2, gin1_b2,
           gin2_W1, gin2_b1, gin2_g, gin2_beta, gin2_W2, gin2_b2,
           fc1_W, fc1_b, bn1_g, bn1_b, fc2_W, fc2_b, bn2_g, bn2_b,
           fc3_W, fc3_b):
    n = x.shape[0]
    f1 = 112
    f2 = 32
    d_self = self_feat.shape[1]
    d_mid = gin2_W2.shape[1]
    d_hid = gin1_W2.shape[1]

    def padv(v, cols, fill=0.0):
        return jnp.pad(v, (0, cols - v.shape[0]),
                       constant_values=fill).reshape(1, cols)

    w1p = _pad_cols(gin1_W1, f1)
    b1p = padv(gin1_b1, f1)
    g1p = padv(gin1_g, f1, 1.0)
    beta1p = padv(gin1_beta, f1)
    w2p = jnp.pad(gin1_W2, ((0, f1 - d_hid), (0, f1 - d_hid)))
    b2p = padv(gin1_b2, f1)
    w12p = jnp.pad(gin2_W1, ((0, f1 - d_hid), (0, f2 - d_mid)))

    b1p2 = padv(gin2_b1, f2)
    g2p = padv(gin2_g, f2, 1.0)
    beta2p = padv(gin2_beta, f2)
    w22p = jnp.pad(gin2_W2, ((0, f2 - d_mid), (0, f2 - d_mid)))
    b2p2 = padv(gin2_b2, f2)

    rmat = np.zeros((f2, d_mid * d_self), np.float32)
    smat = np.zeros((d_self, d_mid * d_self), np.float32)
    for i in range(d_mid):
        for j in range(d_self):
            rmat[i, i * d_self + j] = 1.0
            smat[j, i * d_self + j] = 1.0
    rmat = jnp.asarray(rmat)
    smat = jnp.asarray(smat)

    src2d = edge_index[0].reshape(-1, 100)
    dst2d = edge_index[1].reshape(-1, 100)
    zeros1 = jnp.zeros((_WCH, f1 // 2), jnp.float32)
    zeros2 = jnp.zeros((_WCH, f2), jnp.float32)
    gids = graph_ids.reshape(1, n)

    ystack = _matmul_split(x, w1p)
    acols = _segsum(ystack, src2d, dst2d, zeros1, "col")
    y2 = _mid(ystack, acols, b1p, g1p, beta1p, w2p, b2p, w12p)
    agg2 = _segsum(y2, src2d, dst2d, zeros2, "edge")
    out = _final(y2, agg2, b1p2, g2p, beta2p, w22p, b2p2,
                 gids, self_feat, rmat, smat,
                 fc1_W, fc1_b.reshape(1, -1),
                 bn1_g.reshape(1, -1), bn1_b.reshape(1, -1),
                 fc2_W, fc2_b.reshape(1, -1),
                 bn2_g.reshape(1, -1), bn2_b.reshape(1, -1),
                 fc3_W, fc3_b.reshape(1, -1))
    return out


# speedup vs baseline: 1.1541x; 1.0062x over previous
import functools

import numpy as np
import jax
import jax.numpy as jnp
from jax import lax
from jax.experimental import pallas as pl
from jax.experimental.pallas import tpu as pltpu
from jax.experimental.pallas import tpu_sc as plsc

_EPS = 1e-5


def _mm_body(x_ref, w_ref, o_ref):
    r = jnp.dot(x_ref[...], w_ref[...], preferred_element_type=jnp.float32)
    fh = o_ref.shape[2]
    o_ref[0] = r[:, :fh]
    o_ref[1] = r[:, fh:]


def _matmul_split(x, w):
    return pl.pallas_call(
        _mm_body,
        out_shape=jax.ShapeDtypeStruct((2, x.shape[0], w.shape[1] // 2),
                                       jnp.float32),
    )(x, w)


def _mid_body(ys_ref, agg_ref, b1_ref, g_ref, beta_ref, w2_ref, b2_ref,
              w1n_ref, o_ref):
    n = ys_ref.shape[1]
    y = jnp.concatenate([ys_ref[0], ys_ref[1]], axis=1)
    agg = jnp.concatenate([agg_ref[:n], agg_ref[n:]], axis=1)
    z = y + agg + b1_ref[...]
    mu = jnp.mean(z, axis=0, keepdims=True)
    d = z - mu
    var = jnp.mean(d * d, axis=0, keepdims=True)
    zn = g_ref[...] * d * lax.rsqrt(var + _EPS) + beta_ref[...]
    a = jnp.maximum(zn, 0.0)
    h = jnp.maximum(
        jnp.dot(a, w2_ref[...], preferred_element_type=jnp.float32)
        + b2_ref[...], 0.0)
    o_ref[...] = jnp.dot(h, w1n_ref[...], preferred_element_type=jnp.float32)


def _mid(ys, agg, b1, g, beta, w2, b2, w1n):
    return pl.pallas_call(
        _mid_body,
        out_shape=jax.ShapeDtypeStruct((ys.shape[1], w1n.shape[1]),
                                       jnp.float32),
    )(ys, agg, b1, g, beta, w2, b2, w1n)


def _final_body(y_ref, agg_ref, b1_ref, g_ref, beta_ref, w2_ref, b2_ref,
                gids_ref, sf_ref, r_ref, s_ref,
                fc1w_ref, fc1b_ref, bn1g_ref, bn1b_ref,
                fc2w_ref, fc2b_ref, bn2g_ref, bn2b_ref,
                fc3w_ref, fc3b_ref, o_ref):
    n = y_ref.shape[0]
    z = y_ref[...] + b1_ref[...]
    for p in range(agg_ref.shape[0] // n):
        z = z + agg_ref[p * n:(p + 1) * n, :]
    mu = jnp.mean(z, axis=0, keepdims=True)
    d = z - mu
    var = jnp.mean(d * d, axis=0, keepdims=True)
    zn = g_ref[...] * d * lax.rsqrt(var + _EPS) + beta_ref[...]
    a = jnp.maximum(zn, 0.0)
    h = jnp.maximum(
        jnp.dot(a, w2_ref[...], preferred_element_type=jnp.float32)
        + b2_ref[...], 0.0)

    b_graphs = o_ref.shape[0]
    rows = lax.broadcasted_iota(jnp.int32, (b_graphs, n), 0)
    oh = (rows == gids_ref[...]).astype(jnp.float32)
    sums = jnp.dot(oh, h, preferred_element_type=jnp.float32)
    counts = jnp.sum(oh, axis=1, keepdims=True)
    hg = sums / jnp.maximum(counts, 1.0)

    gmat = (jnp.dot(hg, r_ref[...], preferred_element_type=jnp.float32)
            * jnp.dot(sf_ref[...], s_ref[...],
                      preferred_element_type=jnp.float32))

    def bn(v, gg, bb):
        m = jnp.mean(v, axis=0, keepdims=True)
        dd = v - m
        vv = jnp.mean(dd * dd, axis=0, keepdims=True)
        return gg * dd * lax.rsqrt(vv + _EPS) + bb

    o1 = jnp.maximum(
        bn(jnp.dot(gmat, fc1w_ref[...], preferred_element_type=jnp.float32)
           + fc1b_ref[...], bn1g_ref[...], bn1b_ref[...]), 0.0)
    o2 = jnp.maximum(
        bn(jnp.dot(o1, fc2w_ref[...], preferred_element_type=jnp.float32)
           + fc2b_ref[...], bn2g_ref[...], bn2b_ref[...]), 0.0)
    o_ref[...] = (jnp.dot(o2, fc3w_ref[...],
                          preferred_element_type=jnp.float32)
                  + fc3b_ref[...])


def _final(y, agg, b1, g, beta, w2, b2, gids, sf, r, s,
           fc1w, fc1b, bn1g, bn1b, fc2w, fc2b, bn2g, bn2b, fc3w, fc3b):
    b_graphs = sf.shape[0]
    return pl.pallas_call(
        _final_body,
        out_shape=jax.ShapeDtypeStruct((b_graphs, fc3w.shape[1]),
                                       jnp.float32),
    )(y, agg, b1, g, beta, w2, b2, gids, sf, r, s,
      fc1w, fc1b, bn1g, bn1b, fc2w, fc2b, bn2g, bn2b, fc3w, fc3b)


_NS = 16
_WCH = 80


@functools.lru_cache(maxsize=None)
def _make_segsum(n_nodes, feat, n_edges, mode):
    batch, nb = 100, 10
    if mode == "col":
        bpt = n_edges // _NS // batch
    else:
        bpt = n_edges // (2 * _NS) // batch
    nref = bpt // nb
    assert nref * nb == bpt and nref % 2 == 0
    nwchunk = n_nodes // _WCH
    assert nwchunk * _WCH == n_nodes
    wper = -(-nwchunk // _NS)
    mesh = plsc.VectorSubcoreMesh(core_axis_name="c", subcore_axis_name="s",
                                  num_cores=2)
    scratch = [
        pltpu.VMEM((nb, batch), jnp.int32),
        pltpu.VMEM((nb, batch), jnp.int32),
        pltpu.VMEM((nb, batch), jnp.int32),
        pltpu.VMEM((nb, batch), jnp.int32),
        pltpu.VMEM((batch, feat), jnp.float32),
        pltpu.VMEM((batch, feat), jnp.float32),
        pltpu.VMEM((batch, feat), jnp.float32),
        pltpu.VMEM((batch, feat), jnp.float32),
        pltpu.VMEM_SHARED((n_nodes, feat), jnp.float32),
        pltpu.SemaphoreType.DMA,
        pltpu.SemaphoreType.DMA,
        pltpu.SemaphoreType.DMA,
    ]
    @functools.partial(
        pl.kernel, mesh=mesh,
        compiler_params=pltpu.CompilerParams(use_tc_tiling_on_sc=False),
        out_type=jax.ShapeDtypeStruct((2 * n_nodes, feat), jnp.float32),
        scratch_types=scratch,
    )
    def segsum(y_hbm, src_hbm, dst_hbm, zeros_hbm, out_hbm,
               srcA, dstA, srcB, dstB, r0, r1, r2, r3, acc,
               semg, sems, semi):
        c = lax.axis_index("c")
        s = lax.axis_index("s")
        bufs = (r0, r1, r2, r3)
        if mode == "col":
            ysel = y_hbm.at[c]
            base = s * bpt
        else:
            ysel = y_hbm
            base = (c * _NS + s) * bpt

        pltpu.sync_copy(zeros_hbm, r0.at[pl.ds(0, _WCH)])
        for r in range(wper):
            k = s + _NS * r

            @pl.when(k < nwchunk)
            def _():
                pltpu.sync_copy(r0.at[pl.ds(0, _WCH)],
                                acc.at[pl.ds(k * _WCH, _WCH)])
        plsc.subcore_barrier()

        def idx_start(g, sv, dv):
            blk0 = base + g * nb
            pltpu.async_copy(src_hbm.at[pl.ds(blk0, nb)], sv, semi)
            pltpu.async_copy(dst_hbm.at[pl.ds(blk0, nb)], dv, semi)

        def idx_wait(g, sv, dv):
            blk0 = base + g * nb
            pltpu.make_async_copy(src_hbm.at[pl.ds(blk0, nb)], sv,
                                  semi).wait()
            pltpu.make_async_copy(dst_hbm.at[pl.ds(blk0, nb)], dv,
                                  semi).wait()

        def g_start(sv, j):
            pltpu.async_copy(ysel.at[sv.at[j]], bufs[j % 4], semg)

        def g_wait(sv, j):
            pltpu.make_async_copy(ysel.at[sv.at[j]], bufs[j % 4],
                                  semg).wait()

        def s_start(dv, j):
            pltpu.async_copy(bufs[j % 4], acc.at[dv.at[j]], sems, add=True)

        def s_wait(dv, j):
            pltpu.make_async_copy(bufs[j % 4], acc.at[dv.at[j]],
                                  sems).wait()

        def run_refill(sv, dv):
            nw = nb // 2
            for j in range(4):
                g_start(sv, j)
            for w in range(nw):
                g_wait(sv, 2 * w)
                g_wait(sv, 2 * w + 1)
                s_start(dv, 2 * w)
                s_start(dv, 2 * w + 1)
                if w >= 1:
                    s_wait(dv, 2 * w - 2)
                    s_wait(dv, 2 * w - 1)
                if w + 2 < nw:
                    g_start(sv, 2 * w + 4)
                    g_start(sv, 2 * w + 5)
            s_wait(dv, nb - 2)
            s_wait(dv, nb - 1)

        idx_start(0, srcA, dstA)

        def body(i, carry):
            g0 = 2 * i
            idx_wait(g0, srcA, dstA)
            idx_start(g0 + 1, srcB, dstB)
            run_refill(srcA, dstA)
            idx_wait(g0 + 1, srcB, dstB)

            @pl.when(g0 + 2 < nref)
            def _():
                idx_start(g0 + 2, srcA, dstA)

            run_refill(srcB, dstB)
            return carry

        lax.fori_loop(0, nref // 2, body, 0)
        plsc.subcore_barrier()

        for r in range(wper):
            k = s + _NS * r

            @pl.when(k < nwchunk)
            def _():
                pltpu.sync_copy(acc.at[pl.ds(k * _WCH, _WCH)],
                                r0.at[pl.ds(0, _WCH)])
                pltpu.sync_copy(r0.at[pl.ds(0, _WCH)],
                                out_hbm.at[pl.ds(c * n_nodes + k * _WCH,
                                                 _WCH)])

    return segsum


def _segsum(y, src2d, dst2d, zeros, mode):
    if mode == "col":
        _, n, f = y.shape
    else:
        n, f = y.shape
    e = src2d.shape[0] * src2d.shape[1]
    return _make_segsum(n, f, e, mode)(y, src2d, dst2d, zeros)


def _pad_cols(a, cols):
    return jnp.pad(a, ((0, 0), (0, cols - a.shape[1])))


def kernel(x, edge_index, graph_ids, self_feat,
           gin1_W1, gin1_b1, gin1_g, gin1_beta, gin1_W2, gin1_b2,
           gin2_W1, gin2_b1, gin2_g, gin2_beta, gin2_W2, gin2_b2,
           fc1_W, fc1_b, bn1_g, bn1_b, fc2_W, fc2_b, bn2_g, bn2_b,
           fc3_W, fc3_b):
    n = x.shape[0]
    f1 = 112
    f2 = 32
    d_self = self_feat.shape[1]
    d_mid = gin2_W2.shape[1]
    d_hid = gin1_W2.shape[1]

    def padv(v, cols, fill=0.0):
        return jnp.pad(v, (0, cols - v.shape[0]),
                       constant_values=fill).reshape(1, cols)

    w1p = _pad_cols(gin1_W1, f1)
    b1p = padv(gin1_b1, f1)
    g1p = padv(gin1_g, f1, 1.0)
    beta1p = padv(gin1_beta, f1)
    w2p = jnp.pad(gin1_W2, ((0, f1 - d_hid), (0, f1 - d_hid)))
    b2p = padv(gin1_b2, f1)
    w12p = jnp.pad(gin2_W1, ((0, f1 - d_hid), (0, f2 - d_mid)))

    b1p2 = padv(gin2_b1, f2)
    g2p = padv(gin2_g, f2, 1.0)
    beta2p = padv(gin2_beta, f2)
    w22p = jnp.pad(gin2_W2, ((0, f2 - d_mid), (0, f2 - d_mid)))
    b2p2 = padv(gin2_b2, f2)

    rmat = np.zeros((f2, d_mid * d_self), np.float32)
    smat = np.zeros((d_self, d_mid * d_self), np.float32)
    for i in range(d_mid):
        for j in range(d_self):
            rmat[i, i * d_self + j] = 1.0
            smat[j, i * d_self + j] = 1.0
    rmat = jnp.asarray(rmat)
    smat = jnp.asarray(smat)

    src2d = edge_index[0].reshape(-1, 100)
    dst2d = edge_index[1].reshape(-1, 100)
    zeros1 = jnp.zeros((_WCH, f1 // 2), jnp.float32)
    zeros2 = jnp.zeros((_WCH, f2), jnp.float32)
    gids = graph_ids.reshape(1, n)

    ystack = _matmul_split(x, w1p)
    acols = _segsum(ystack, src2d, dst2d, zeros1, "col")
    y2 = _mid(ystack, acols, b1p, g1p, beta1p, w2p, b2p, w12p)
    agg2 = _segsum(y2, src2d, dst2d, zeros2, "edge")
    out = _final(y2, agg2, b1p2, g2p, beta2p, w22p, b2p2,
                 gids, self_feat, rmat, smat,
                 fc1_W, fc1_b.reshape(1, -1),
                 bn1_g.reshape(1, -1), bn1_b.reshape(1, -1),
                 fc2_W, fc2_b.reshape(1, -1),
                 bn2_g.reshape(1, -1), bn2_b.reshape(1, -1),
                 fc3_W, fc3_b.reshape(1, -1))
    return out


# speedup vs baseline: 1.1598x; 1.0050x over previous
import functools

import numpy as np
import jax
import jax.numpy as jnp
from jax import lax
from jax.experimental import pallas as pl
from jax.experimental.pallas import tpu as pltpu
from jax.experimental.pallas import tpu_sc as plsc

_EPS = 1e-5


def _mm_body(x_ref, w_ref, o_ref):
    n = x_ref.shape[0]
    fh = o_ref.shape[2]
    r = jnp.dot(x_ref[...], w_ref[...], preferred_element_type=jnp.float32)
    o_ref[0] = r[:, :fh]
    pad = 2 * fh - w_ref.shape[1]
    o_ref[1] = jnp.concatenate(
        [r[:, fh:], jnp.zeros((n, pad), jnp.float32)], axis=1)


def _matmul_split(x, w, fh):
    return pl.pallas_call(
        _mm_body,
        out_shape=jax.ShapeDtypeStruct((2, x.shape[0], fh), jnp.float32),
    )(x, w)


def _pad_row(v, width, fill=0.0):
    k = v.shape[0]
    return jnp.concatenate(
        [v.reshape(1, k), jnp.full((1, width - k), fill, jnp.float32)],
        axis=1)


def _mid_body(ys_ref, agg_ref, b1_ref, g_ref, beta_ref, w2_ref, b2_ref,
              w1n_ref, o_ref):
    n = ys_ref.shape[1]
    fw = 2 * ys_ref.shape[2]
    dh = w2_ref.shape[0]
    y = jnp.concatenate([ys_ref[0], ys_ref[1]], axis=1)
    agg = jnp.concatenate([agg_ref[:n], agg_ref[n:]], axis=1)
    z = y + agg + _pad_row(b1_ref[...], fw)
    mu = jnp.mean(z, axis=0, keepdims=True)
    d = z - mu
    var = jnp.mean(d * d, axis=0, keepdims=True)
    zn = (_pad_row(g_ref[...], fw, 1.0) * d * lax.rsqrt(var + _EPS)
          + _pad_row(beta_ref[...], fw))
    a = jnp.maximum(zn, 0.0)[:, :dh]
    h = jnp.maximum(
        jnp.dot(a, w2_ref[...], preferred_element_type=jnp.float32)
        + b2_ref[...].reshape(1, dh), 0.0)
    r = jnp.dot(h, w1n_ref[...], preferred_element_type=jnp.float32)
    o_ref[...] = jnp.concatenate(
        [r, jnp.zeros((n, o_ref.shape[1] - r.shape[1]), jnp.float32)],
        axis=1)


def _mid(ys, agg, b1, g, beta, w2, b2, w1n, f2):
    return pl.pallas_call(
        _mid_body,
        out_shape=jax.ShapeDtypeStruct((ys.shape[1], f2), jnp.float32),
    )(ys, agg, b1, g, beta, w2, b2, w1n)


def _final_body(y_ref, agg_ref, b1_ref, g_ref, beta_ref, w2_ref, b2_ref,
                gids_ref, sf_ref, r_ref, s_ref,
                fc1w_ref, fc1b_ref, bn1g_ref, bn1b_ref,
                fc2w_ref, fc2b_ref, bn2g_ref, bn2b_ref,
                fc3w_ref, fc3b_ref, o_ref):
    n = y_ref.shape[0]
    fw = y_ref.shape[1]
    dm = w2_ref.shape[0]
    z = y_ref[...] + _pad_row(b1_ref[...], fw)
    for p in range(agg_ref.shape[0] // n):
        z = z + agg_ref[p * n:(p + 1) * n, :]
    mu = jnp.mean(z, axis=0, keepdims=True)
    d = z - mu
    var = jnp.mean(d * d, axis=0, keepdims=True)
    zn = (_pad_row(g_ref[...], fw, 1.0) * d * lax.rsqrt(var + _EPS)
          + _pad_row(beta_ref[...], fw))
    a = jnp.maximum(zn, 0.0)[:, :dm]
    h = jnp.maximum(
        jnp.dot(a, w2_ref[...], preferred_element_type=jnp.float32)
        + b2_ref[...].reshape(1, dm), 0.0)

    b_graphs = o_ref.shape[0]
    rows = lax.broadcasted_iota(jnp.int32, (b_graphs, n), 0)
    oh = (rows == gids_ref[...].reshape(1, n)).astype(jnp.float32)
    sums = jnp.dot(oh, h, preferred_element_type=jnp.float32)
    counts = jnp.sum(oh, axis=1, keepdims=True)
    hg = sums / jnp.maximum(counts, 1.0)

    gmat = (jnp.dot(hg, r_ref[...], preferred_element_type=jnp.float32)
            * jnp.dot(sf_ref[...], s_ref[...],
                      preferred_element_type=jnp.float32))

    def bn(v, gg, bb):
        m = jnp.mean(v, axis=0, keepdims=True)
        dd = v - m
        vv = jnp.mean(dd * dd, axis=0, keepdims=True)
        k = gg.shape[0]
        return (gg.reshape(1, k) * dd * lax.rsqrt(vv + _EPS)
                + bb.reshape(1, k))

    o1 = jnp.maximum(
        bn(jnp.dot(gmat, fc1w_ref[...], preferred_element_type=jnp.float32)
           + fc1b_ref[...].reshape(1, -1), bn1g_ref[...], bn1b_ref[...]),
        0.0)
    o2 = jnp.maximum(
        bn(jnp.dot(o1, fc2w_ref[...], preferred_element_type=jnp.float32)
           + fc2b_ref[...].reshape(1, -1), bn2g_ref[...], bn2b_ref[...]),
        0.0)
    o_ref[...] = (jnp.dot(o2, fc3w_ref[...],
                          preferred_element_type=jnp.float32)
                  + fc3b_ref[...].reshape(1, -1))


def _final(y, agg, b1, g, beta, w2, b2, gids, sf, r, s,
           fc1w, fc1b, bn1g, bn1b, fc2w, fc2b, bn2g, bn2b, fc3w, fc3b):
    b_graphs = sf.shape[0]
    return pl.pallas_call(
        _final_body,
        out_shape=jax.ShapeDtypeStruct((b_graphs, fc3w.shape[1]),
                                       jnp.float32),
    )(y, agg, b1, g, beta, w2, b2, gids, sf, r, s,
      fc1w, fc1b, bn1g, bn1b, fc2w, fc2b, bn2g, bn2b, fc3w, fc3b)


_NS = 16
_WCH = 80


@functools.lru_cache(maxsize=None)
def _make_segsum(n_nodes, feat, n_edges, mode):
    batch, nb = 100, 10
    if mode == "col":
        bpt = n_edges // _NS // batch
    else:
        bpt = n_edges // (2 * _NS) // batch
    nref = bpt // nb
    assert nref * nb == bpt and nref % 2 == 0
    nwchunk = n_nodes // _WCH
    assert nwchunk * _WCH == n_nodes
    wper = -(-nwchunk // _NS)
    mesh = plsc.VectorSubcoreMesh(core_axis_name="c", subcore_axis_name="s",
                                  num_cores=2)
    scratch = [
        pltpu.VMEM((nb, batch), jnp.int32),
        pltpu.VMEM((nb, batch), jnp.int32),
        pltpu.VMEM((nb, batch), jnp.int32),
        pltpu.VMEM((nb, batch), jnp.int32),
        pltpu.VMEM((batch, feat), jnp.float32),
        pltpu.VMEM((batch, feat), jnp.float32),
        pltpu.VMEM((batch, feat), jnp.float32),
        pltpu.VMEM((batch, feat), jnp.float32),
        pltpu.VMEM_SHARED((n_nodes, feat), jnp.float32),
        pltpu.SemaphoreType.DMA,
        pltpu.SemaphoreType.DMA,
        pltpu.SemaphoreType.DMA,
    ]
    @functools.partial(
        pl.kernel, mesh=mesh,
        compiler_params=pltpu.CompilerParams(use_tc_tiling_on_sc=False),
        out_type=jax.ShapeDtypeStruct((2 * n_nodes, feat), jnp.float32),
        scratch_types=scratch,
    )
    def segsum(y_hbm, src_hbm, dst_hbm, zeros_hbm, out_hbm,
               srcA, dstA, srcB, dstB, r0, r1, r2, r3, acc,
               semg, sems, semi):
        c = lax.axis_index("c")
        s = lax.axis_index("s")
        bufs = (r0, r1, r2, r3)
        if mode == "col":
            ysel = y_hbm.at[c]
            base = s * bpt
        else:
            ysel = y_hbm
            base = (c * _NS + s) * bpt

        pltpu.sync_copy(zeros_hbm, r0.at[pl.ds(0, _WCH)])
        for r in range(wper):
            k = s + _NS * r

            @pl.when(k < nwchunk)
            def _():
                pltpu.sync_copy(r0.at[pl.ds(0, _WCH)],
                                acc.at[pl.ds(k * _WCH, _WCH)])
        plsc.subcore_barrier()

        def idx_start(g, sv, dv):
            blk0 = base + g * nb
            pltpu.async_copy(src_hbm.at[pl.ds(blk0, nb)], sv, semi)
            pltpu.async_copy(dst_hbm.at[pl.ds(blk0, nb)], dv, semi)

        def idx_wait(g, sv, dv):
            blk0 = base + g * nb
            pltpu.make_async_copy(src_hbm.at[pl.ds(blk0, nb)], sv,
                                  semi).wait()
            pltpu.make_async_copy(dst_hbm.at[pl.ds(blk0, nb)], dv,
                                  semi).wait()

        def g_start(sv, j):
            pltpu.async_copy(ysel.at[sv.at[j]], bufs[j % 4], semg)

        def g_wait(sv, j):
            pltpu.make_async_copy(ysel.at[sv.at[j]], bufs[j % 4],
                                  semg).wait()

        def s_start(dv, j):
            pltpu.async_copy(bufs[j % 4], acc.at[dv.at[j]], sems, add=True)

        def s_wait(dv, j):
            pltpu.make_async_copy(bufs[j % 4], acc.at[dv.at[j]],
                                  sems).wait()

        def run_refill(sv, dv):
            nw = nb // 2
            for j in range(4):
                g_start(sv, j)
            for w in range(nw):
                g_wait(sv, 2 * w)
                g_wait(sv, 2 * w + 1)
                s_start(dv, 2 * w)
                s_start(dv, 2 * w + 1)
                if w >= 1:
                    s_wait(dv, 2 * w - 2)
                    s_wait(dv, 2 * w - 1)
                if w + 2 < nw:
                    g_start(sv, 2 * w + 4)
                    g_start(sv, 2 * w + 5)
            s_wait(dv, nb - 2)
            s_wait(dv, nb - 1)

        idx_start(0, srcA, dstA)

        def body(i, carry):
            g0 = 2 * i
            idx_wait(g0, srcA, dstA)
            idx_start(g0 + 1, srcB, dstB)
            run_refill(srcA, dstA)
            idx_wait(g0 + 1, srcB, dstB)

            @pl.when(g0 + 2 < nref)
            def _():
                idx_start(g0 + 2, srcA, dstA)

            run_refill(srcB, dstB)
            return carry

        lax.fori_loop(0, nref // 2, body, 0)
        plsc.subcore_barrier()

        for r in range(wper):
            k = s + _NS * r

            @pl.when(k < nwchunk)
            def _():
                pltpu.sync_copy(acc.at[pl.ds(k * _WCH, _WCH)],
                                r0.at[pl.ds(0, _WCH)])
                pltpu.sync_copy(r0.at[pl.ds(0, _WCH)],
                                out_hbm.at[pl.ds(c * n_nodes + k * _WCH,
                                                 _WCH)])

    return segsum


def _segsum(y, src2d, dst2d, zeros, mode):
    if mode == "col":
        _, n, f = y.shape
    else:
        n, f = y.shape
    e = src2d.shape[0] * src2d.shape[1]
    return _make_segsum(n, f, e, mode)(y, src2d, dst2d, zeros)


def kernel(x, edge_index, graph_ids, self_feat,
           gin1_W1, gin1_b1, gin1_g, gin1_beta, gin1_W2, gin1_b2,
           gin2_W1, gin2_b1, gin2_g, gin2_beta, gin2_W2, gin2_b2,
           fc1_W, fc1_b, bn1_g, bn1_b, fc2_W, fc2_b, bn2_g, bn2_b,
           fc3_W, fc3_b):
    n = x.shape[0]
    f1 = 112
    f2 = 32
    d_self = self_feat.shape[1]
    d_mid = gin2_W2.shape[1]

    rmat = np.zeros((d_mid, d_mid * d_self), np.float32)
    smat = np.zeros((d_self, d_mid * d_self), np.float32)
    for i in range(d_mid):
        for j in range(d_self):
            rmat[i, i * d_self + j] = 1.0
            smat[j, i * d_self + j] = 1.0
    rmat = jnp.asarray(rmat)
    smat = jnp.asarray(smat)

    src2d = edge_index[0].reshape(-1, 100)
    dst2d = edge_index[1].reshape(-1, 100)
    zeros1 = jnp.zeros((_WCH, f1 // 2), jnp.float32)
    zeros2 = jnp.zeros((_WCH, f2), jnp.float32)

    ystack = _matmul_split(x, gin1_W1, f1 // 2)
    acols = _segsum(ystack, src2d, dst2d, zeros1, "col")
    y2 = _mid(ystack, acols, gin1_b1, gin1_g, gin1_beta,
              gin1_W2, gin1_b2, gin2_W1, f2)
    agg2 = _segsum(y2, src2d, dst2d, zeros2, "edge")
    out = _final(y2, agg2, gin2_b1, gin2_g, gin2_beta, gin2_W2, gin2_b2,
                 graph_ids, self_feat, rmat, smat,
                 fc1_W, fc1_b, bn1_g, bn1_b,
                 fc2_W, fc2_b, bn2_g, bn2_b, fc3_W, fc3_b)
    return out
